# Initial kernel scaffold; baseline (speedup 1.0000x reference)
import jax
import jax.numpy as jnp
from jax.experimental import pallas as pl


def kernel(x, edge_index, Ws, bs, Wc, bc):
    raise NotImplementedError("write your pallas kernel here")


# speedup vs baseline: 8.2353x; 8.2353x over previous
import functools

import jax
import jax.numpy as jnp
from jax import lax
from jax.experimental import pallas as pl
from jax.experimental.pallas import tpu as pltpu
from jax.experimental.pallas import tpu_sc as plsc

NC = 2
NS = 16
CH = 128
WD = 16
BN = 2000


def _sc_scatter_make(N, D, NCH):
    n_tab = -(-(N + 1) // (NS * 16)) * (NS * 16)
    rows_z = n_tab // NS
    mesh = plsc.VectorSubcoreMesh(core_axis_name="c", subcore_axis_name="s",
                                  num_cores=NC, num_subcores=NS)

    @functools.partial(
        pl.kernel,
        out_type=jax.ShapeDtypeStruct((NC, n_tab, D), jnp.float32),
        mesh=mesh,
        scratch_types=[
            pltpu.VMEM((2, CH), jnp.int32),
            pltpu.VMEM((CH, D), jnp.float32),
            pltpu.VMEM((16, D), jnp.float32),
            pltpu.VMEM_SHARED((n_tab, D), jnp.float32),
            pltpu.SemaphoreType.DMA,
        ],
    )
    def k(t_hbm, ei_hbm, out_hbm, idx_v, gbuf, zbuf, agg, sem):
        c = lax.axis_index("c")
        s = lax.axis_index("s")
        for i in range(16):
            for j in range(D // 16):
                zbuf[i, pl.ds(j * 16, 16)] = jnp.zeros((16,), jnp.float32)

        def zrow(kk, carry):
            pltpu.sync_copy(zbuf, agg.at[pl.ds(s * rows_z + kk * 16, 16)])
            return carry
        lax.fori_loop(0, rows_z // 16, zrow, 0)
        plsc.subcore_barrier()

        base = (c * NS + s) * NCH

        def chunk(j, carry):
            pltpu.sync_copy(ei_hbm.at[base + j], idx_v)
            pltpu.async_copy(t_hbm.at[idx_v.at[0]], gbuf, sem).wait()
            pltpu.sync_copy(gbuf, agg.at[idx_v.at[1]], add=True)
            return carry
        lax.fori_loop(0, NCH, chunk, 0)
        plsc.subcore_barrier()

        pltpu.sync_copy(agg.at[pl.ds(s * rows_z, rows_z)],
                        out_hbm.at[c, pl.ds(s * rows_z, rows_z)])

    return k


def _sc_deg_make(N, NCH):
    n_tab = -(-(N + 1) // (NS * 16)) * (NS * 16)
    rows_z = n_tab // NS
    mesh = plsc.VectorSubcoreMesh(core_axis_name="c", subcore_axis_name="s",
                                  num_cores=NC, num_subcores=NS)

    @functools.partial(
        pl.kernel,
        out_type=jax.ShapeDtypeStruct((NC, n_tab, WD), jnp.float32),
        mesh=mesh,
        scratch_types=[
            pltpu.VMEM((2, CH), jnp.int32),
            pltpu.VMEM((CH, WD), jnp.float32),
            pltpu.VMEM((16, WD), jnp.float32),
            pltpu.VMEM_SHARED((n_tab, WD), jnp.float32),
            pltpu.SemaphoreType.DMA,
        ],
    )
    def k(ei_hbm, out_hbm, idx_v, obuf, zbuf, deg, sem):
        c = lax.axis_index("c")
        s = lax.axis_index("s")
        one0 = jnp.where(lax.iota(jnp.int32, 16) == 0,
                         jnp.float32(1.0), jnp.float32(0.0))
        for i in range(CH):
            obuf[i, pl.ds(0, 16)] = one0
        for i in range(16):
            zbuf[i, pl.ds(0, 16)] = jnp.zeros((16,), jnp.float32)

        def zrow(kk, carry):
            pltpu.sync_copy(zbuf, deg.at[pl.ds(s * rows_z + kk * 16, 16)])
            return carry
        lax.fori_loop(0, rows_z // 16, zrow, 0)
        plsc.subcore_barrier()

        base = (c * NS + s) * NCH

        def chunk(j, carry):
            pltpu.sync_copy(ei_hbm.at[base + j], idx_v)
            pltpu.sync_copy(obuf, deg.at[idx_v.at[1]], add=True)
            return carry
        lax.fori_loop(0, NCH, chunk, 0)
        plsc.subcore_barrier()

        pltpu.sync_copy(deg.at[pl.ds(s * rows_z, rows_z)],
                        out_hbm.at[c, pl.ds(s * rows_z, rows_z)])

    return k


def _dinv_of(d0, d1):
    return lax.rsqrt(jnp.sum(d0 + d1, axis=1, keepdims=True) + 1.0)


def _tc_first_body(d0, d1, x, w, o):
    dinv = _dinv_of(d0[...], d1[...])
    o[...] = dinv * jnp.dot(x[...], w[...], preferred_element_type=jnp.float32)


def _tc_mid_body(d0, d1, s0, s1, t, b, w, o):
    dinv = _dinv_of(d0[...], d1[...])
    h = jnp.maximum(dinv * (s0[...] + s1[...] + t[...]) + b[...], 0.0)
    o[...] = dinv * jnp.dot(h, w[...], preferred_element_type=jnp.float32)


def _tc_last_body(d0, d1, s0, s1, t, b, x, wc, bcp, o):
    dinv = _dinv_of(d0[...], d1[...])
    h = jnp.maximum(dinv * (s0[...] + s1[...] + t[...]) + b[...], 0.0)
    o[...] = jnp.dot(h + x[...], wc[...],
                     preferred_element_type=jnp.float32) + bcp[...]


def _half_spec(cols, half):
    return pl.BlockSpec((None, BN, cols), lambda i, _h=half: (_h, i, 0))


def _row_spec(cols):
    return pl.BlockSpec((BN, cols), lambda i: (i, 0))


def _full_spec(r, c):
    return pl.BlockSpec((r, c), lambda i: (0, 0))


def _tc_first(deg2, x, w, N, D):
    return pl.pallas_call(
        _tc_first_body,
        grid=(N // BN,),
        in_specs=[_half_spec(WD, 0), _half_spec(WD, 1),
                  _row_spec(D), _full_spec(D, D)],
        out_specs=_row_spec(D),
        out_shape=jax.ShapeDtypeStruct((N, D), jnp.float32),
    )(deg2, deg2, x, w)


def _tc_mid(deg2, s2, t, b, w, N, D):
    return pl.pallas_call(
        _tc_mid_body,
        grid=(N // BN,),
        in_specs=[_half_spec(WD, 0), _half_spec(WD, 1),
                  _half_spec(D, 0), _half_spec(D, 1),
                  _row_spec(D), _full_spec(1, D), _full_spec(D, D)],
        out_specs=_row_spec(D),
        out_shape=jax.ShapeDtypeStruct((N, D), jnp.float32),
    )(deg2, deg2, s2, s2, t, b, w)


def _tc_last(deg2, s2, t, b, x, wcp, bcp, N, D):
    return pl.pallas_call(
        _tc_last_body,
        grid=(N // BN,),
        in_specs=[_half_spec(WD, 0), _half_spec(WD, 1),
                  _half_spec(D, 0), _half_spec(D, 1),
                  _row_spec(D), _full_spec(1, D),
                  _row_spec(D), _full_spec(D, D), _full_spec(1, D)],
        out_specs=_row_spec(D),
        out_shape=jax.ShapeDtypeStruct((N, D), jnp.float32),
    )(deg2, deg2, s2, s2, t, b, x, wcp, bcp)


def kernel(x, edge_index, Ws, bs, Wc, bc):
    N, D = x.shape
    E = edge_index.shape[1]
    L = Ws.shape[0]
    OUT = Wc.shape[1]

    NCH = -(-E // (NC * NS * CH))
    e_pad = NC * NS * CH * NCH
    src = jnp.concatenate(
        [edge_index[0], jnp.zeros((e_pad - E,), jnp.int32)])
    dst = jnp.concatenate(
        [edge_index[1], jnp.full((e_pad - E,), N, jnp.int32)])
    ei_chunks = (jnp.stack([src, dst])
                 .reshape(2, NC * NS * NCH, CH)
                 .transpose(1, 0, 2))

    sc_deg = _sc_deg_make(N, NCH)
    sc_scatter = _sc_scatter_make(N, D, NCH)

    deg2 = sc_deg(ei_chunks)
    t = _tc_first(deg2, x, Ws[0], N, D)
    for i in range(1, L):
        s2 = sc_scatter(t, ei_chunks)
        t = _tc_mid(deg2, s2, t, bs[i - 1].reshape(1, D), Ws[i], N, D)
    s2 = sc_scatter(t, ei_chunks)

    wcp = jnp.zeros((D, D), jnp.float32).at[:, :OUT].set(Wc)
    bcp = jnp.zeros((1, D), jnp.float32).at[0, :OUT].set(bc)
    out_p = _tc_last(deg2, s2, t, bs[L - 1].reshape(1, D), x, wcp, bcp, N, D)
    return out_p[:, :OUT]
